# Initial kernel scaffold; baseline (speedup 1.0000x reference)
import jax
import jax.numpy as jnp
from jax.experimental import pallas as pl


def kernel(inputs, mask, embed, W, b):
    raise NotImplementedError("write your pallas kernel here")


# speedup vs baseline: 3.8245x; 3.8245x over previous
import functools

import jax
import jax.numpy as jnp
from jax import lax
from jax.experimental import pallas as pl
from jax.experimental.pallas import tpu as pltpu
from jax.experimental.pallas import tpu_sc as plsc

NC, NS, LANES = 2, 16, 16
NW = NC * NS

B, L = 16384, 50
D = 32
RPW = B // NW
CHUNKS = RPW // LANES
KROWS = (RPW * L) // 128


def _dot_body(e_ref, w_ref, o_ref):
    o_ref[...] = jnp.sum(e_ref[...] * w_ref[...][None], axis=2)


def _precompute_s(embed, W):
    V = embed.shape[0]
    C = 1000
    R = V // C
    RB = 8
    e3 = embed.reshape(R, C, D)
    s2 = pl.pallas_call(
        _dot_body,
        grid=(R // RB,),
        in_specs=[
            pl.BlockSpec((RB, C, D), lambda i: (i, 0, 0)),
            pl.BlockSpec((1, D), lambda i: (0, 0)),
        ],
        out_specs=pl.BlockSpec((RB, C), lambda i: (i, 0)),
        out_shape=jax.ShapeDtypeStruct((R, C), jnp.float32),
    )(e3, W)
    return s2.reshape(V)


def _sc_body(s_hbm, idx_hbm, mask_hbm, b_hbm, out_hbm,
             idx_v, mask_v, vals_v, out_v, b_v, sem):
    wid = lax.axis_index("s") * NC + lax.axis_index("c")
    pltpu.sync_copy(idx_hbm.at[wid], idx_v)
    pltpu.sync_copy(mask_hbm.at[wid], mask_v)
    pltpu.sync_copy(b_hbm, b_v)
    pltpu.async_copy(s_hbm.at[idx_v], vals_v, sem).wait()

    bias = b_v[...]
    zero = jnp.zeros((LANES,), jnp.float32)
    for c in range(CHUNKS):
        def body(l, carry, c=c):
            acc, msum = carry
            off = l * RPW + c * LANES
            v = vals_v[pl.ds(off, LANES)]
            m = mask_v[pl.ds(off, LANES)]
            return acc + v * m, msum + m
        acc, msum = lax.fori_loop(0, L, body, (zero, zero))
        out_v[pl.ds(c * LANES, LANES)] = acc / (msum + 1e-9) + bias
    pltpu.sync_copy(out_v, out_hbm.at[pl.ds(wid * RPW, RPW)])


@functools.cache
def _make_sc_call():
    mesh = plsc.VectorSubcoreMesh(
        core_axis_name="c", subcore_axis_name="s",
        num_cores=NC, num_subcores=NS)
    return pl.kernel(
        _sc_body,
        out_type=jax.ShapeDtypeStruct((B,), jnp.float32),
        mesh=mesh,
        scratch_types=[
            pltpu.VMEM((RPW * L,), jnp.int32),
            pltpu.VMEM((RPW * L,), jnp.float32),
            pltpu.VMEM((RPW * L,), jnp.float32),
            pltpu.VMEM((RPW,), jnp.float32),
            pltpu.VMEM((LANES,), jnp.float32),
            pltpu.SemaphoreType.DMA,
        ],
    )


@jax.jit
def kernel(inputs, mask, embed, W, b):
    s = _precompute_s(embed.astype(jnp.float32), W.astype(jnp.float32))
    idx_t = (inputs.astype(jnp.int32)
             .reshape(NW, RPW, L).swapaxes(1, 2).reshape(NW, RPW * L))
    mask_t = (mask.astype(jnp.float32)
              .reshape(NW, RPW, L).swapaxes(1, 2).reshape(NW, RPW * L))
    b16 = jnp.broadcast_to(b.astype(jnp.float32).reshape(()), (LANES,))
    return _make_sc_call()(s, idx_t, mask_t, b16)
